# Initial kernel scaffold; baseline (speedup 1.0000x reference)
import jax
import jax.numpy as jnp
from jax.experimental import pallas as pl


def kernel(scores, drugset_mul_hot):
    raise NotImplementedError("write your pallas kernel here")


# speedup vs baseline: 1.3119x; 1.3119x over previous
import jax
import jax.numpy as jnp
from jax.experimental import pallas as pl


_B, _N = 16384, 128
_R = 32


def _body(s_ref, d_ref, out_ref):
    s = s_ref[...]
    d = d_ref[...]

    a = s[:, :, None]
    b = s[:, None, :]
    t = jnp.sum((a > b).astype(jnp.float32), axis=1)

    io = jax.lax.broadcasted_iota(
        jnp.int32, (_R, _N, _N), 1).astype(jnp.float32)
    cmp = (t[:, None, :] <= io).astype(jnp.float32)
    inter = jnp.sum(cmp * d[:, None, :], axis=2)
    mtot = jnp.sum(cmp, axis=2)

    ell = jnp.sum(d, axis=1, keepdims=True)
    jac = inter / (mtot + ell - inter)

    maxv = jnp.max(jac, axis=1, keepdims=True)
    lane = jax.lax.broadcasted_iota(jnp.int32, (_R, _N), 1)
    idx = jnp.min(jnp.where(jac >= maxv, lane, _N), axis=1)

    out = jnp.concatenate(
        [(idx + 1)[:, None], ell.astype(jnp.int32)], axis=1)
    out_ref[...] = out


@jax.jit
def kernel(scores, drugset_mul_hot):
    d = drugset_mul_hot.astype(jnp.float32)
    grid = (_B // _R,)
    return pl.pallas_call(
        _body,
        grid=grid,
        in_specs=[
            pl.BlockSpec((_R, _N), lambda i: (i, 0)),
            pl.BlockSpec((_R, _N), lambda i: (i, 0)),
        ],
        out_specs=pl.BlockSpec((_R, 2), lambda i: (i, 0)),
        out_shape=jax.ShapeDtypeStruct((_B, 2), jnp.int32),
    )(scores, d)
